# Initial kernel scaffold; baseline (speedup 1.0000x reference)
import jax
import jax.numpy as jnp
from jax.experimental import pallas as pl


def kernel(x0, x1, x2, x3, emb_word, emb_bigram, emb_trigram, W1, b1, W2, b2):
    raise NotImplementedError("write your pallas kernel here")


# speedup vs baseline: 1.2902x; 1.2902x over previous
import functools

import jax
import jax.numpy as jnp
from jax import lax
from jax.experimental import pallas as pl
from jax.experimental.pallas import tpu as pltpu
from jax.experimental.pallas import tpu_sc as plsc

_B = 4096
_L = 50
_E = 200
_HID = 256
_NCLS = 20

_NC = 2
_NS = 16
_NW = _NC * _NS
_BPW = _B // _NW
_K = 128
_NCHUNK = (_BPW * _L) // _K


def _sc_pool_body(x0r, x2r, x3r, dest_hbm, zeros_hbm, tw, tb, tt,
                  ow, ob, ot,
                  idx_v, dest_v, buf, accw, accb, acct, sem):
    c = lax.axis_index("c")
    s = lax.axis_index("s")
    w = c * _NS + s
    base = w * _BPW

    pltpu.sync_copy(dest_hbm.at[s], dest_v)

    pltpu.sync_copy(zeros_hbm, buf)
    for acc in (accw, accb, acct):
        pltpu.sync_copy(buf, acc.at[pl.ds(s * _BPW, _BPW)])

    for xr, tab, acc in ((x0r, tw, accw), (x2r, tb, accb), (x3r, tt, acct)):
        pltpu.sync_copy(xr.at[w], idx_v)

        def body(j, carry, tab=tab, acc=acc):
            pltpu.async_copy(tab.at[idx_v.at[j]], buf, sem).wait()
            pltpu.sync_copy(buf, acc.at[dest_v.at[j]], add=True)
            return carry

        lax.fori_loop(0, _NCHUNK, body, 0)

    for acc, out in ((accw, ow), (accb, ob), (acct, ot)):
        pltpu.sync_copy(acc.at[pl.ds(s * _BPW, _BPW)], buf)
        pltpu.sync_copy(buf, out.at[pl.ds(base, _BPW)])


@jax.jit
def _sc_pool(x0r, x2r, x3r, dest, zeros, emb_word, emb_bigram, emb_trigram):
    mesh = plsc.VectorSubcoreMesh(core_axis_name="c", subcore_axis_name="s")
    shape = jax.ShapeDtypeStruct((_B, _E), jnp.float32)
    return pl.kernel(
        _sc_pool_body,
        out_type=(shape, shape, shape),
        mesh=mesh,
        scratch_types=[
            pltpu.VMEM((_NCHUNK, _K), jnp.int32),
            pltpu.VMEM((_NCHUNK, _K), jnp.int32),
            pltpu.VMEM((_K, _E), jnp.float32),
            pltpu.VMEM_SHARED((_NS * _BPW, _E), jnp.float32),
            pltpu.VMEM_SHARED((_NS * _BPW, _E), jnp.float32),
            pltpu.VMEM_SHARED((_NS * _BPW, _E), jnp.float32),
            pltpu.SemaphoreType.DMA,
        ],
        compiler_params=pltpu.CompilerParams(use_tc_tiling_on_sc=False),
    )(x0r, x2r, x3r, dest, zeros, emb_word, emb_bigram, emb_trigram)


def _mlp_body(pw, pb, pt, w1w, w1b, w1t, b1r, w2r, b2r, out):
    h = jnp.dot(pw[...], w1w[...], preferred_element_type=jnp.float32)
    h += jnp.dot(pb[...], w1b[...], preferred_element_type=jnp.float32)
    h += jnp.dot(pt[...], w1t[...], preferred_element_type=jnp.float32)
    h = h * (1.0 / _L) + b1r[...]
    h = jnp.maximum(h, 0.0)
    out[...] = jnp.dot(h, w2r[...], preferred_element_type=jnp.float32) + b2r[...]


_BB = 1024


@jax.jit
def _mlp(pw, pb, pt, W1, b1, W2, b2):
    w1w, w1b, w1t = W1[:_E], W1[_E:2 * _E], W1[2 * _E:]
    grid = (_B // _BB,)
    blk = pl.BlockSpec((_BB, _E), lambda i: (i, 0))
    full = lambda r, ccols: pl.BlockSpec((r, ccols), lambda i: (0, 0))
    return pl.pallas_call(
        _mlp_body,
        grid=grid,
        in_specs=[blk, blk, blk,
                  full(_E, _HID), full(_E, _HID), full(_E, _HID),
                  full(1, _HID), full(_HID, _NCLS), full(1, _NCLS)],
        out_specs=pl.BlockSpec((_BB, _NCLS), lambda i: (i, 0)),
        out_shape=jax.ShapeDtypeStruct((_B, _NCLS), jnp.float32),
    )(pw, pb, pt, w1w, w1b, w1t, b1.reshape(1, _HID), W2, b2.reshape(1, _NCLS))


def kernel(x0, x1, x2, x3, emb_word, emb_bigram, emb_trigram, W1, b1, W2, b2):
    del x1
    x0r = x0.reshape(_NW, _NCHUNK, _K)
    x2r = x2.reshape(_NW, _NCHUNK, _K)
    x3r = x3.reshape(_NW, _NCHUNK, _K)
    pos = (jnp.arange(_BPW * _L, dtype=jnp.int32) // _L).reshape(_NCHUNK, _K)
    dest = jnp.arange(_NS, dtype=jnp.int32)[:, None, None] * _BPW + pos[None]
    zeros = jnp.zeros((_K, _E), jnp.float32)
    pw, pb, pt = _sc_pool(x0r, x2r, x3r, dest, zeros,
                          emb_word, emb_bigram, emb_trigram)
    return _mlp(pw, pb, pt, W1, b1, W2, b2)


# speedup vs baseline: 3.2004x; 2.4805x over previous
import functools

import jax
import jax.numpy as jnp
from jax import lax
from jax.experimental import pallas as pl
from jax.experimental.pallas import tpu as pltpu
from jax.experimental.pallas import tpu_sc as plsc

_B = 4096
_L = 50
_E = 200
_EA = 128
_EB = _E - _EA
_HID = 256
_NCLS = 20

_NC = 2
_NS = 16
_NW = _NC * _NS
_BPW = _B // _NW
_K = 128
_NCHUNK = (_BPW * _L) // _K

_RV = 2048


def _tail_body(tab_ref, out_ref):
    lane = lax.broadcasted_iota(jnp.int32, (_RV, _EA), 1)
    out_ref[...] = jnp.where(lane < _EB, tab_ref[...], 0.0)


@jax.jit
def _tails(emb_word, emb_bigram, emb_trigram):
    outs = []
    for tab in (emb_word, emb_bigram, emb_trigram):
        v = tab.shape[0]
        grid = (pl.cdiv(v, _RV),)
        outs.append(pl.pallas_call(
            _tail_body,
            grid=grid,
            in_specs=[pl.BlockSpec((_RV, _EA), lambda i: (i, 1))],
            out_specs=pl.BlockSpec((_RV, _EA), lambda i: (i, 0)),
            out_shape=jax.ShapeDtypeStruct((v, _EA), jnp.float32),
        )(tab))
    return outs


def _sc_pool_body(x0r, x2r, x3r, dest_hbm, zeros_hbm,
                  tw, tb, tt, tlw, tlb, tlt,
                  owa, owb, oba, obb, ota, otb,
                  idx_v, dest_v, bufa, bufb, acca, accb, sema, semb):
    c = lax.axis_index("c")
    s = lax.axis_index("s")
    w = c * _NS + s
    base = w * _BPW

    pltpu.sync_copy(dest_hbm.at[s], dest_v)

    tabs = ((x0r, tw, tlw, owa, owb),
            (x2r, tb, tlb, oba, obb),
            (x3r, tt, tlt, ota, otb))
    for xr, tab, tail, oa, ob in tabs:
        pltpu.sync_copy(xr.at[w], idx_v)

        pltpu.sync_copy(zeros_hbm, bufa)
        pltpu.sync_copy(bufa, acca.at[pl.ds(s * _BPW, _BPW)])
        pltpu.sync_copy(bufa, accb.at[pl.ds(s * _BPW, _BPW)])

        def body(j, carry, tab=tab, tail=tail):
            cpa = pltpu.async_copy(tab.at[idx_v.at[j], pl.ds(0, _EA)],
                                   bufa, sema)
            cpb = pltpu.async_copy(tail.at[idx_v.at[j]], bufb, semb)
            cpa.wait()
            pltpu.sync_copy(bufa, acca.at[dest_v.at[j]], add=True)
            cpb.wait()
            pltpu.sync_copy(bufb, accb.at[dest_v.at[j]], add=True)
            return carry

        lax.fori_loop(0, _NCHUNK, body, 0)

        pltpu.sync_copy(acca.at[pl.ds(s * _BPW, _BPW)], bufa)
        pltpu.sync_copy(bufa, oa.at[pl.ds(base, _BPW)])
        pltpu.sync_copy(accb.at[pl.ds(s * _BPW, _BPW)], bufb)
        pltpu.sync_copy(bufb, ob.at[pl.ds(base, _BPW)])


@jax.jit
def _sc_pool(x0r, x2r, x3r, dest, zeros,
             emb_word, emb_bigram, emb_trigram, tail_w, tail_b, tail_t):
    mesh = plsc.VectorSubcoreMesh(core_axis_name="c", subcore_axis_name="s")
    sa = jax.ShapeDtypeStruct((_B, _EA), jnp.float32)
    return pl.kernel(
        _sc_pool_body,
        out_type=(sa,) * 6,
        mesh=mesh,
        scratch_types=[
            pltpu.VMEM((_NCHUNK, _K), jnp.int32),
            pltpu.VMEM((_NCHUNK, _K), jnp.int32),
            pltpu.VMEM((_K, _EA), jnp.float32),
            pltpu.VMEM((_K, _EA), jnp.float32),
            pltpu.VMEM_SHARED((_NS * _BPW, _EA), jnp.float32),
            pltpu.VMEM_SHARED((_NS * _BPW, _EA), jnp.float32),
            pltpu.SemaphoreType.DMA,
            pltpu.SemaphoreType.DMA,
        ],
        compiler_params=pltpu.CompilerParams(use_tc_tiling_on_sc=True),
    )(x0r, x2r, x3r, dest, zeros,
      emb_word, emb_bigram, emb_trigram, tail_w, tail_b, tail_t)


def _mlp_body(pwa, pwb, pba, pbb, pta, ptb,
              w1wa, w1wb, w1ba, w1bb, w1ta, w1tb,
              b1r, w2r, b2r, out):
    h = jnp.dot(pwa[...], w1wa[...], preferred_element_type=jnp.float32)
    h += jnp.dot(pwb[...], w1wb[...], preferred_element_type=jnp.float32)
    h += jnp.dot(pba[...], w1ba[...], preferred_element_type=jnp.float32)
    h += jnp.dot(pbb[...], w1bb[...], preferred_element_type=jnp.float32)
    h += jnp.dot(pta[...], w1ta[...], preferred_element_type=jnp.float32)
    h += jnp.dot(ptb[...], w1tb[...], preferred_element_type=jnp.float32)
    h = h * (1.0 / _L) + b1r[...]
    h = jnp.maximum(h, 0.0)
    out[...] = jnp.dot(h, w2r[...], preferred_element_type=jnp.float32) + b2r[...]


_BB = 1024


@jax.jit
def _mlp(pwa, pwb, pba, pbb, pta, ptb, W1, b1, W2, b2):
    pad = jnp.zeros((_EA - _EB, _HID), jnp.float32)
    w1a = [W1[t * _E:t * _E + _EA] for t in range(3)]
    w1b_ = [jnp.concatenate([W1[t * _E + _EA:(t + 1) * _E], pad])
            for t in range(3)]
    grid = (_B // _BB,)
    blk = pl.BlockSpec((_BB, _EA), lambda i: (i, 0))
    full = lambda r, ccols: pl.BlockSpec((r, ccols), lambda i: (0, 0))
    return pl.pallas_call(
        _mlp_body,
        grid=grid,
        in_specs=[blk] * 6 + [full(_EA, _HID)] * 6
                 + [full(1, _HID), full(_HID, _NCLS), full(1, _NCLS)],
        out_specs=pl.BlockSpec((_BB, _NCLS), lambda i: (i, 0)),
        out_shape=jax.ShapeDtypeStruct((_B, _NCLS), jnp.float32),
    )(pwa, pwb, pba, pbb, pta, ptb,
      w1a[0], w1b_[0], w1a[1], w1b_[1], w1a[2], w1b_[2],
      b1.reshape(1, _HID), W2, b2.reshape(1, _NCLS))


def kernel(x0, x1, x2, x3, emb_word, emb_bigram, emb_trigram, W1, b1, W2, b2):
    del x1
    x0r = x0.reshape(_NW, _NCHUNK, _K)
    x2r = x2.reshape(_NW, _NCHUNK, _K)
    x3r = x3.reshape(_NW, _NCHUNK, _K)
    pos = (jnp.arange(_BPW * _L, dtype=jnp.int32) // _L).reshape(_NCHUNK, _K)
    dest = jnp.arange(_NS, dtype=jnp.int32)[:, None, None] * _BPW + pos[None]
    zeros = jnp.zeros((_K, _EA), jnp.float32)
    tail_w, tail_b, tail_t = _tails(emb_word, emb_bigram, emb_trigram)
    pwa, pwb, pba, pbb, pta, ptb = _sc_pool(
        x0r, x2r, x3r, dest, zeros,
        emb_word, emb_bigram, emb_trigram, tail_w, tail_b, tail_t)
    return _mlp(pwa, pwb, pba, pbb, pta, ptb, W1, b1, W2, b2)


# speedup vs baseline: 3.3713x; 1.0534x over previous
import functools

import jax
import jax.numpy as jnp
from jax import lax
from jax.experimental import pallas as pl
from jax.experimental.pallas import tpu as pltpu
from jax.experimental.pallas import tpu_sc as plsc

_B = 4096
_L = 50
_E = 200
_EA = 128
_EB = _E - _EA
_HID = 256
_NCLS = 20

_NC = 2
_NS = 16
_NW = _NC * _NS
_BPW = _B // _NW
_K = 128
_NCHUNK = (_BPW * _L) // _K

_RV = 2048


def _tail_body(tab_ref, out_ref):
    tail = tab_ref[:, _EA:_E]
    out_ref[...] = jnp.concatenate(
        [tail, jnp.zeros((_RV, _EA - _EB), jnp.float32)], axis=1)


@jax.jit
def _tails(emb_word, emb_bigram, emb_trigram):
    outs = []
    for tab in (emb_word, emb_bigram, emb_trigram):
        v = tab.shape[0]
        grid = (pl.cdiv(v, _RV),)
        outs.append(pl.pallas_call(
            _tail_body,
            grid=grid,
            in_specs=[pl.BlockSpec((_RV, _E), lambda i: (i, 0))],
            out_specs=pl.BlockSpec((_RV, _EA), lambda i: (i, 0)),
            out_shape=jax.ShapeDtypeStruct((v, _EA), jnp.float32),
        )(tab))
    return outs


def _sc_pool_body(x0r, x2r, x3r, dest_hbm, zeros_hbm,
                  tw, tb, tt, tlw, tlb, tlt,
                  owa, owb, oba, obb, ota, otb,
                  idx_v, dest_v, bufa, bufb, acca, accb, sema, semb):
    c = lax.axis_index("c")
    s = lax.axis_index("s")
    w = c * _NS + s
    base = w * _BPW

    pltpu.sync_copy(dest_hbm.at[s], dest_v)

    tabs = ((x0r, tw, tlw, owa, owb),
            (x2r, tb, tlb, oba, obb),
            (x3r, tt, tlt, ota, otb))
    for xr, tab, tail, oa, ob in tabs:
        pltpu.sync_copy(xr.at[w], idx_v)

        pltpu.sync_copy(zeros_hbm, bufa.at[0])
        pltpu.sync_copy(bufa.at[0], acca.at[pl.ds(s * _BPW, _BPW)])
        pltpu.sync_copy(bufa.at[0], accb.at[pl.ds(s * _BPW, _BPW)])

        def fire(j, p, tab=tab, tail=tail):
            cpa = pltpu.async_copy(tab.at[idx_v.at[j], pl.ds(0, _EA)],
                                   bufa.at[p], sema.at[p])
            cpb = pltpu.async_copy(tail.at[idx_v.at[j]],
                                   bufb.at[p], semb.at[p])
            return cpa, cpb

        fire(0, 0)

        def body(j, carry, tab=tab, tail=tail):
            p = lax.rem(j, 2)
            fire(j, p, tab, tail)
            pj = j - 1
            pp = lax.rem(pj, 2)
            cpa = pltpu.make_async_copy(
                tab.at[idx_v.at[pj], pl.ds(0, _EA)], bufa.at[pp], sema.at[pp])
            cpb = pltpu.make_async_copy(
                tail.at[idx_v.at[pj]], bufb.at[pp], semb.at[pp])
            cpa.wait()
            pltpu.sync_copy(bufa.at[pp], acca.at[dest_v.at[pj]], add=True)
            cpb.wait()
            pltpu.sync_copy(bufb.at[pp], accb.at[dest_v.at[pj]], add=True)
            return carry

        lax.fori_loop(1, _NCHUNK, body, 0)

        lastp = (_NCHUNK - 1) % 2
        cpa = pltpu.make_async_copy(
            tab.at[idx_v.at[_NCHUNK - 1], pl.ds(0, _EA)],
            bufa.at[lastp], sema.at[lastp])
        cpb = pltpu.make_async_copy(
            tail.at[idx_v.at[_NCHUNK - 1]], bufb.at[lastp], semb.at[lastp])
        cpa.wait()
        pltpu.sync_copy(bufa.at[lastp], acca.at[dest_v.at[_NCHUNK - 1]],
                        add=True)
        cpb.wait()
        pltpu.sync_copy(bufb.at[lastp], accb.at[dest_v.at[_NCHUNK - 1]],
                        add=True)

        pltpu.sync_copy(acca.at[pl.ds(s * _BPW, _BPW)], bufa.at[0])
        pltpu.sync_copy(bufa.at[0], oa.at[pl.ds(base, _BPW)])
        pltpu.sync_copy(accb.at[pl.ds(s * _BPW, _BPW)], bufb.at[0])
        pltpu.sync_copy(bufb.at[0], ob.at[pl.ds(base, _BPW)])


@jax.jit
def _sc_pool(x0r, x2r, x3r, dest, zeros,
             emb_word, emb_bigram, emb_trigram, tail_w, tail_b, tail_t):
    mesh = plsc.VectorSubcoreMesh(core_axis_name="c", subcore_axis_name="s")
    sa = jax.ShapeDtypeStruct((_B, _EA), jnp.float32)
    return pl.kernel(
        _sc_pool_body,
        out_type=(sa,) * 6,
        mesh=mesh,
        scratch_types=[
            pltpu.VMEM((_NCHUNK, _K), jnp.int32),
            pltpu.VMEM((_NCHUNK, _K), jnp.int32),
            pltpu.VMEM((2, _K, _EA), jnp.float32),
            pltpu.VMEM((2, _K, _EA), jnp.float32),
            pltpu.VMEM_SHARED((_NS * _BPW, _EA), jnp.float32),
            pltpu.VMEM_SHARED((_NS * _BPW, _EA), jnp.float32),
            pltpu.SemaphoreType.DMA((2,)),
            pltpu.SemaphoreType.DMA((2,)),
        ],
        compiler_params=pltpu.CompilerParams(use_tc_tiling_on_sc=True),
    )(x0r, x2r, x3r, dest, zeros,
      emb_word, emb_bigram, emb_trigram, tail_w, tail_b, tail_t)


def _mlp_body(pwa, pwb, pba, pbb, pta, ptb,
              w1wa, w1wb, w1ba, w1bb, w1ta, w1tb,
              b1r, w2r, b2r, out):
    h = jnp.dot(pwa[...], w1wa[...], preferred_element_type=jnp.float32)
    h += jnp.dot(pwb[...], w1wb[...], preferred_element_type=jnp.float32)
    h += jnp.dot(pba[...], w1ba[...], preferred_element_type=jnp.float32)
    h += jnp.dot(pbb[...], w1bb[...], preferred_element_type=jnp.float32)
    h += jnp.dot(pta[...], w1ta[...], preferred_element_type=jnp.float32)
    h += jnp.dot(ptb[...], w1tb[...], preferred_element_type=jnp.float32)
    h = h * (1.0 / _L) + b1r[...]
    h = jnp.maximum(h, 0.0)
    out[...] = jnp.dot(h, w2r[...], preferred_element_type=jnp.float32) + b2r[...]


_BB = 1024


@jax.jit
def _mlp(pwa, pwb, pba, pbb, pta, ptb, W1, b1, W2, b2):
    pad = jnp.zeros((_EA - _EB, _HID), jnp.float32)
    w1a = [W1[t * _E:t * _E + _EA] for t in range(3)]
    w1b_ = [jnp.concatenate([W1[t * _E + _EA:(t + 1) * _E], pad])
            for t in range(3)]
    grid = (_B // _BB,)
    blk = pl.BlockSpec((_BB, _EA), lambda i: (i, 0))
    full = lambda r, ccols: pl.BlockSpec((r, ccols), lambda i: (0, 0))
    return pl.pallas_call(
        _mlp_body,
        grid=grid,
        in_specs=[blk] * 6 + [full(_EA, _HID)] * 6
                 + [full(1, _HID), full(_HID, _NCLS), full(1, _NCLS)],
        out_specs=pl.BlockSpec((_BB, _NCLS), lambda i: (i, 0)),
        out_shape=jax.ShapeDtypeStruct((_B, _NCLS), jnp.float32),
    )(pwa, pwb, pba, pbb, pta, ptb,
      w1a[0], w1b_[0], w1a[1], w1b_[1], w1a[2], w1b_[2],
      b1.reshape(1, _HID), W2, b2.reshape(1, _NCLS))


def kernel(x0, x1, x2, x3, emb_word, emb_bigram, emb_trigram, W1, b1, W2, b2):
    del x1
    x0r = x0.reshape(_NW, _NCHUNK, _K)
    x2r = x2.reshape(_NW, _NCHUNK, _K)
    x3r = x3.reshape(_NW, _NCHUNK, _K)
    pos = (jnp.arange(_BPW * _L, dtype=jnp.int32) // _L).reshape(_NCHUNK, _K)
    dest = jnp.arange(_NS, dtype=jnp.int32)[:, None, None] * _BPW + pos[None]
    zeros = jnp.zeros((_K, _EA), jnp.float32)
    tail_w, tail_b, tail_t = _tails(emb_word, emb_bigram, emb_trigram)
    pwa, pwb, pba, pbb, pta, ptb = _sc_pool(
        x0r, x2r, x3r, dest, zeros,
        emb_word, emb_bigram, emb_trigram, tail_w, tail_b, tail_t)
    return _mlp(pwa, pwb, pba, pbb, pta, ptb, W1, b1, W2, b2)
